# baseline (speedup 1.0000x reference)
import functools

import jax
import jax.numpy as jnp
from jax import lax
from jax.experimental import pallas as pl
from jax.experimental.pallas import tpu as pltpu

_FM_H = 19
_FM_W = 19
_HW = _FM_H * _FM_W
_SCORE_FILTER_SIZE = (3, 2)
_SCORE_FILTER_NUM = (3, 3)
_PATCH_SIZE = (224, 112)
_NEG = float("-inf")


def _shift(x, o, fill):
    if o == 0:
        return x
    n = x.shape[1]
    f = jnp.full((1, abs(o)), fill, x.dtype)
    if o > 0:
        return jnp.concatenate([x[:, o:], f], axis=1)
    return jnp.concatenate([f, x[:, : n + o]], axis=1)


def _nms_body(scale_sref, fm_ref, loc_ref, val_ref, *, c_chunk):
    b = pl.program_id(0)
    n_chunks = fm_ref.shape[1] // c_chunk

    def step(c, acc):
        return acc + fm_ref[0, pl.ds(c * c_chunk, c_chunk), :]

    acc = lax.fori_loop(0, n_chunks, step,
                        jnp.zeros((c_chunk, _HW), jnp.float32))
    s_flat = jnp.sum(acc, axis=0, keepdims=True)

    p_i = lax.broadcasted_iota(jnp.int32, (1, _HW), 1)
    row = p_i // _FM_W
    col = p_i % _FM_W

    sh = scale_sref[2 * b]
    sw = scale_sref[2 * b + 1]
    smin = jnp.minimum(sh, sw)
    sb_h = (sh - smin) // 2
    sb_w = (sw - smin) // 2
    smin_f = smin.astype(jnp.float32)
    sb_hf = sb_h.astype(jnp.float32)
    sb_wf = sb_w.astype(jnp.float32)

    loc_mat = jnp.zeros((1, 6, 4), jnp.int32)
    val_vec = jnp.zeros((1, 1, 6), jnp.float32)
    ri6 = lax.broadcasted_iota(jnp.int32, (1, 6, 4), 1)
    ci4 = lax.broadcasted_iota(jnp.int32, (1, 6, 4), 2)
    ci6 = lax.broadcasted_iota(jnp.int32, (1, 1, 6), 2)

    m = 0
    for t in range(2):
        k = _SCORE_FILTER_SIZE[t]
        ps = _PATCH_SIZE[t]
        s = _FM_H - k + 1
        pooled = jnp.zeros((1, _HW), jnp.float32)
        for di in range(k):
            for dj in range(k):
                pooled = pooled + _shift(s_flat, di * _FM_W + dj, 0.0)
        pooled = pooled / float(k * k)
        valid = (row < s) & (col < s)
        score = jnp.where(valid, pooled, _NEG)
        for _f in range(_SCORE_FILTER_NUM[t]):
            vmax = jnp.max(score)
            psel = jnp.min(jnp.where(score == vmax, p_i, _HW))
            tp = jnp.full((1, _HW), _NEG, jnp.float32)
            for di in (-1, 0, 1):
                for dj in (-1, 0, 1):
                    nb = _shift(score, di * _FM_W + dj, _NEG)
                    okc = (col + dj >= 0) & (col + dj < _FM_W)
                    tp = jnp.maximum(tp, jnp.where(okc, nb, _NEG))
            score = jnp.where((tp == vmax) & valid, 0.0, score)

            i = psel // _FM_W
            j = psel % _FM_W
            rate_h = (2.0 * i.astype(jnp.float32) + float(_FM_H - s + 1)) / (2.0 * _FM_H)
            rate_w = (2.0 * j.astype(jnp.float32) + float(_FM_W - s + 1)) / (2.0 * _FM_W)
            c_h = (sb_hf + smin_f * rate_h).astype(jnp.int32)
            c_w = (sb_wf + smin_f * rate_w).astype(jnp.int32)
            top = c_h - ps // 2
            bot = c_h + ps // 2 + ps % 2
            lef = c_w - ps // 2
            rig = c_w + ps // 2 + ps % 2
            below_h = jnp.minimum(top, 0)
            top = top - below_h
            bot = bot - below_h
            below_w = jnp.minimum(lef, 0)
            lef = lef - below_w
            rig = rig - below_w
            over_h = jnp.maximum(bot - sh, 0)
            top = jnp.maximum(top - over_h, 0)
            bot = bot - over_h
            over_w = jnp.maximum(rig - sw, 0)
            lef = jnp.maximum(lef - over_w, 0)
            rig = rig - over_w

            rowvals = (jnp.where(ci4 == 0, top, 0)
                       + jnp.where(ci4 == 1, lef, 0)
                       + jnp.where(ci4 == 2, bot, 0)
                       + jnp.where(ci4 == 3, rig, 0))
            loc_mat = loc_mat + jnp.where(ri6 == m, rowvals, 0)
            val_vec = val_vec + jnp.where(ci6 == m, vmax, 0.0)
            m += 1

    loc_ref[0] = loc_mat
    val_ref[0] = val_vec


@jax.jit
def kernel(fm, scale):
    b, c, h, w = fm.shape
    fm2 = fm.reshape(b, c, h * w)
    scale_flat = scale.reshape(b * 2)
    c_chunk = 8

    grid_spec = pltpu.PrefetchScalarGridSpec(
        num_scalar_prefetch=1,
        grid=(b,),
        in_specs=[pl.BlockSpec((1, c, h * w), lambda bb, sref: (bb, 0, 0))],
        out_specs=[
            pl.BlockSpec((1, 1, 6, 4), lambda bb, sref: (bb, 0, 0, 0)),
            pl.BlockSpec((1, 1, 1, 6), lambda bb, sref: (bb, 0, 0, 0)),
        ],
    )
    loc, vals = pl.pallas_call(
        functools.partial(_nms_body, c_chunk=c_chunk),
        grid_spec=grid_spec,
        out_shape=[
            jax.ShapeDtypeStruct((b, 1, 6, 4), jnp.int32),
            jax.ShapeDtypeStruct((b, 1, 1, 6), jnp.float32),
        ],
    )(scale_flat, fm2)
    return loc.reshape(b, 6, 4), vals.reshape(b, 6)


# speedup vs baseline: 1.4067x; 1.4067x over previous
import functools

import jax
import jax.numpy as jnp
from jax import lax
from jax.experimental import pallas as pl
from jax.experimental.pallas import tpu as pltpu

_FM_H = 19
_FM_W = 19
_HW = _FM_H * _FM_W
_SCORE_FILTER_SIZE = (3, 2)
_SCORE_FILTER_NUM = (3, 3)
_PATCH_SIZE = (224, 112)
_NEG = float("-inf")


def _shift(x, o, fill):
    if o == 0:
        return x
    n = x.shape[1]
    f = jnp.full((x.shape[0], abs(o)), fill, x.dtype)
    if o > 0:
        return jnp.concatenate([x[:, o:], f], axis=1)
    return jnp.concatenate([f, x[:, : n + o]], axis=1)


def _body(fm_ref, scale_ref, loc_ref, val_ref, s_scratch, *, c_chunk):
    b = pl.program_id(0)
    nb = pl.num_programs(0)
    n_chunks = fm_ref.shape[1] // c_chunk

    def step(c, acc):
        return acc + fm_ref[0, pl.ds(c * c_chunk, c_chunk), :]

    acc = lax.fori_loop(0, n_chunks, step,
                        jnp.zeros((c_chunk, _HW), jnp.float32))
    s_scratch[pl.ds(b, 1), :] = jnp.sum(acc, axis=0, keepdims=True)

    @pl.when(b == nb - 1)
    def _nms():
        bsz = s_scratch.shape[0]
        s_all = s_scratch[...]
        p_i = lax.broadcasted_iota(jnp.int32, (bsz, _HW), 1)
        row = p_i // _FM_W
        col = p_i % _FM_W

        sh = scale_ref[:, 0:1]
        sw = scale_ref[:, 1:2]
        smin = jnp.minimum(sh, sw)
        sb_hf = ((sh - smin) // 2).astype(jnp.float32)
        sb_wf = ((sw - smin) // 2).astype(jnp.float32)
        smin_f = smin.astype(jnp.float32)

        loc_cols = []
        val_cols = []
        for t in range(2):
            k = _SCORE_FILTER_SIZE[t]
            ps = _PATCH_SIZE[t]
            s = _FM_H - k + 1
            pooled = jnp.zeros((bsz, _HW), jnp.float32)
            for di in range(k):
                for dj in range(k):
                    pooled = pooled + _shift(s_all, di * _FM_W + dj, 0.0)
            pooled = pooled / float(k * k)
            valid = (row < s) & (col < s)
            score = jnp.where(valid, pooled, _NEG)
            for _f in range(_SCORE_FILTER_NUM[t]):
                vmax = jnp.max(score, axis=1, keepdims=True)
                psel = jnp.min(jnp.where(score == vmax, p_i, _HW),
                               axis=1, keepdims=True)
                tp = jnp.full((bsz, _HW), _NEG, jnp.float32)
                for di in (-1, 0, 1):
                    for dj in (-1, 0, 1):
                        nb_ = _shift(score, di * _FM_W + dj, _NEG)
                        okc = (col + dj >= 0) & (col + dj < _FM_W)
                        tp = jnp.maximum(tp, jnp.where(okc, nb_, _NEG))
                score = jnp.where((tp == vmax) & valid, 0.0, score)

                i = psel // _FM_W
                j = psel % _FM_W
                rate_h = (2.0 * i.astype(jnp.float32) + float(_FM_H - s + 1)) / (2.0 * _FM_H)
                rate_w = (2.0 * j.astype(jnp.float32) + float(_FM_W - s + 1)) / (2.0 * _FM_W)
                c_h = (sb_hf + smin_f * rate_h).astype(jnp.int32)
                c_w = (sb_wf + smin_f * rate_w).astype(jnp.int32)
                top = c_h - ps // 2
                bot = c_h + ps // 2 + ps % 2
                lef = c_w - ps // 2
                rig = c_w + ps // 2 + ps % 2
                below_h = jnp.minimum(top, 0)
                top = top - below_h
                bot = bot - below_h
                below_w = jnp.minimum(lef, 0)
                lef = lef - below_w
                rig = rig - below_w
                over_h = jnp.maximum(bot - sh, 0)
                top = jnp.maximum(top - over_h, 0)
                bot = bot - over_h
                over_w = jnp.maximum(rig - sw, 0)
                lef = jnp.maximum(lef - over_w, 0)
                rig = rig - over_w
                loc_cols.append(jnp.concatenate([top, lef, bot, rig], axis=1))
                val_cols.append(vmax)

        loc_ref[...] = jnp.concatenate(loc_cols, axis=1)
        val_ref[...] = jnp.concatenate(val_cols, axis=1)


@jax.jit
def kernel(fm, scale):
    b, c, h, w = fm.shape
    fm2 = fm.reshape(b, c, h * w)
    c_chunk = 32

    loc, vals = pl.pallas_call(
        functools.partial(_body, c_chunk=c_chunk),
        grid=(b,),
        in_specs=[
            pl.BlockSpec((1, c, h * w), lambda bb: (bb, 0, 0)),
            pl.BlockSpec((b, 2), lambda bb: (0, 0)),
        ],
        out_specs=[
            pl.BlockSpec((b, 24), lambda bb: (0, 0)),
            pl.BlockSpec((b, 6), lambda bb: (0, 0)),
        ],
        out_shape=[
            jax.ShapeDtypeStruct((b, 24), jnp.int32),
            jax.ShapeDtypeStruct((b, 6), jnp.float32),
        ],
        scratch_shapes=[pltpu.VMEM((b, _HW), jnp.float32)],
    )(fm2, scale)
    return loc.reshape(b, 6, 4), vals.reshape(b, 6)
